# Initial kernel scaffold; baseline (speedup 1.0000x reference)
import jax
import jax.numpy as jnp
from jax.experimental import pallas as pl


def kernel(input_ids, codebooks, codes):
    raise NotImplementedError("write your pallas kernel here")


# speedup vs baseline: 3.5226x; 3.5226x over previous
import functools

import jax
import jax.numpy as jnp
from jax import lax
from jax.experimental import pallas as pl
from jax.experimental.pallas import tpu as pltpu
from jax.experimental.pallas import tpu_sc as plsc

N = 100000
D = 128
M = 4
K = 256
SUB = D // M
CPAD = 16

NC = 2
NS = 16
NW = NC * NS
L16 = 16

BL = 4096 * 50
PER_TILE = BL // NW
SB = 128
NSB = PER_TILE // SB
QG = SB * M // L16


def _pq_kernel(ids_hbm, cb_hbm, codes_hbm, out_hbm,
               ids_v, cb_v, c16_v0, c16_v1, cflat_v, out_v0, out_v1,
               sem_c0, sem_c1, sem_o0, sem_o1):
    wid = lax.axis_index("s") * NC + lax.axis_index("c")
    row0 = wid * PER_TILE

    pltpu.sync_copy(ids_hbm.at[wid], ids_v)

    iota = lax.iota(jnp.int32, L16)
    lane_lt4 = iota < 4
    lane_m = iota & 3
    lane_id = iota >> 2
    mvec = lane_m * (K * SUB)
    packoff = lane_id * D + lane_m * SUB

    c16 = (c16_v0, c16_v1)
    sem_c = (sem_c0, sem_c1)
    outs = (out_v0, out_v1)
    sem_o = (sem_o0, sem_o1)

    def fire_codes(sb, b):
        pltpu.async_copy(codes_hbm.at[ids_v.at[sb]], c16[b], sem_c[b])

    def wait_codes(sb, b):
        pltpu.make_async_copy(codes_hbm.at[ids_v.at[sb]], c16[b],
                              sem_c[b]).wait()

    fire_codes(0, 0)
    fire_codes(1, 1)
    pltpu.sync_copy(cb_hbm, cb_v)

    def do_block(sb, b):
        wait_codes(sb, b)

        def deint(r, carry):
            v = c16[b][r]
            plsc.store_scatter(cflat_v, [r * 4 + iota], v)
            return carry
        lax.fori_loop(0, SB, deint, 0)

        @pl.when(sb + 2 < NSB)
        def _refill():
            pltpu.async_copy(codes_hbm.at[ids_v.at[sb + 2]], c16[b], sem_c[b])

        @pl.when(sb >= 2)
        def _wait_out():
            pltpu.make_async_copy(
                outs[b], out_hbm.at[pl.ds(row0 * D, SB * D)], sem_o[b]).wait()

        def group(q, carry):
            cvec = cflat_v[pl.ds(q * L16, L16)]
            base = cvec * SUB + mvec
            st_base = q * 4 * D + packoff
            for t in range(SUB):
                val = plsc.load_gather(cb_v, [base + t])
                plsc.store_scatter(outs[b], [st_base + t], val)
            return carry
        lax.fori_loop(0, QG, group, 0)

        pltpu.async_copy(
            outs[b], out_hbm.at[pl.ds((row0 + sb * SB) * D, SB * D)],
            sem_o[b])

    def outer(o, carry):
        do_block(o * 2, 0)
        do_block(o * 2 + 1, 1)
        return carry

    lax.fori_loop(0, NSB // 2, outer, 0)

    pltpu.make_async_copy(
        out_v0, out_hbm.at[pl.ds(row0 * D, SB * D)], sem_o0).wait()
    pltpu.make_async_copy(
        out_v1, out_hbm.at[pl.ds(row0 * D, SB * D)], sem_o1).wait()


@jax.jit
def _pq_embedding(ids3d, cb_flat, codes16):
    k = functools.partial(
        pl.kernel,
        mesh=plsc.VectorSubcoreMesh(core_axis_name="c", subcore_axis_name="s"),
        out_type=jax.ShapeDtypeStruct((BL * D,), jnp.float32),
        scratch_types=[
            pltpu.VMEM((NSB, 128), jnp.int32),
            pltpu.VMEM((M * K * SUB,), jnp.float32),
            pltpu.VMEM((SB, CPAD), jnp.int32),
            pltpu.VMEM((SB, CPAD), jnp.int32),
            pltpu.VMEM((SB * M + L16 - M,), jnp.int32),
            pltpu.VMEM((SB * D,), jnp.float32),
            pltpu.VMEM((SB * D,), jnp.float32),
            pltpu.SemaphoreType.DMA,
            pltpu.SemaphoreType.DMA,
            pltpu.SemaphoreType.DMA,
            pltpu.SemaphoreType.DMA,
        ],
        compiler_params=pltpu.CompilerParams(needs_layout_passes=False,
                                             use_tc_tiling_on_sc=False),
    )(_pq_kernel)
    return k(ids3d, cb_flat, codes16)


def kernel(input_ids, codebooks, codes):
    B, Lseq = input_ids.shape
    ids3d = input_ids.reshape(NW, NSB, 128)
    cb_flat = codebooks.reshape(-1)
    codes16 = jnp.pad(codes, ((0, 0), (0, CPAD - M)))
    out = _pq_embedding(ids3d, cb_flat, codes16)
    return out.reshape(B, Lseq, D)
